# baseline (speedup 1.0000x reference)
import functools

import jax
import jax.numpy as jnp
from jax import lax
from jax.experimental import pallas as pl
from jax.experimental.pallas import tpu as pltpu
from jax.experimental.pallas import tpu_sc as plsc

H_DIM = 64
BATCH = 16384

_S_SC = 8192
_B_TC = BATCH - _S_SC

_NC = 2
_NS = 16
_NW = _NC * _NS
_B_PER_W = _S_SC // _NW
_G = 16
_N_G = _B_PER_W // _G
_HP = 16
_N_HP = H_DIM // _HP

_GT = 16
_N_GT = _B_TC // _GT
_GPP = 8
_TC_GRID = _N_GT // _GPP


def _emb_body(idx_hbm, table_hbm, out_hbm, idx_v, staged_v, out_v, sem0, sem1):
    wid = lax.axis_index("s") * _NC + lax.axis_index("c")
    base = wid * _B_PER_W
    pltpu.sync_copy(idx_hbm.at[pl.ds(base, _B_PER_W)], idx_v)

    lanes = lax.iota(jnp.int32, 16)
    sems = (sem0, sem1)

    def starts_of(g):
        r = idx_v[pl.ds(g * _G, _G)]
        a128 = lax.shift_left(lax.shift_right_logical(r, 7), 7)
        return [pl.multiple_of(a128[k], 128) for k in range(_G)]

    def issue(starts, h, buf):
        for k in range(_G):
            pltpu.async_copy(
                table_hbm.at[pl.ds(h * _HP, _HP), pl.ds(starts[k], 128)],
                staged_v.at[buf, k],
                sems[buf],
            )

    def drain(buf):
        for _ in range(_G):
            pltpu.make_async_copy(
                table_hbm.at[pl.ds(0, _HP), pl.ds(0, 128)],
                staged_v.at[buf, 0],
                sems[buf],
            ).wait()

    def extract(l128, g, h, buf):
        for cc in range(_HP):
            vals = plsc.load_gather(
                staged_v.at[buf], [lanes, jnp.full((16,), cc, jnp.int32), l128]
            )
            out_v[h * _HP + cc, pl.ds(g * _G, _G)] = vals

    issue(starts_of(0), 0, 0)

    def _group(g, carry):
        r = idx_v[pl.ds(g * _G, _G)]
        l128 = lax.bitwise_and(r, 127)
        starts = starts_of(g)
        issue(starts, 1, 1)
        drain(0)
        extract(l128, g, 0, 0)
        issue(starts, 2, 0)
        drain(1)
        extract(l128, g, 1, 1)
        issue(starts, 3, 1)
        drain(0)
        extract(l128, g, 2, 0)
        g_next = lax.min(g + 1, _N_G - 1)
        issue(starts_of(g_next), 0, 0)
        drain(1)
        extract(l128, g, 3, 1)
        return carry

    lax.fori_loop(0, _N_G, _group, 0)
    drain(0)
    pltpu.sync_copy(out_v, out_hbm.at[:, pl.ds(base, _B_PER_W)])


def _sc_gather(idx_sc, table_t):
    f = functools.partial(
        pl.kernel,
        mesh=plsc.VectorSubcoreMesh(core_axis_name="c", subcore_axis_name="s"),
        out_type=jax.ShapeDtypeStruct((H_DIM, _S_SC), jnp.float32),
        compiler_params=pltpu.CompilerParams(needs_layout_passes=False),
        scratch_types=[
            pltpu.VMEM((_B_PER_W,), jnp.int32),
            pltpu.VMEM((2, _G, _HP, 128), jnp.float32),
            pltpu.VMEM((H_DIM, _B_PER_W), jnp.float32),
            pltpu.SemaphoreType.DMA,
            pltpu.SemaphoreType.DMA,
        ],
    )(_emb_body)
    return f(idx_sc, table_t)


def _tc_body(idx_smem, idx2d, table_hbm, out_ref, stage, sem0, sem1):
    j = pl.program_id(0)
    sems = (sem0, sem1)

    def issue(g, buf):
        for k in range(_GT):
            a = lax.shift_left(lax.shift_right_logical(idx_smem[g * _GT + k], 7), 7)
            pltpu.make_async_copy(
                table_hbm.at[:, pl.ds(pl.multiple_of(a, 128), 128)],
                stage.at[buf, :, pl.ds(k * 128, 128)],
                sems[buf],
            ).start()

    def drain(buf):
        pltpu.make_async_copy(
            table_hbm.at[:, pl.ds(0, 128 * _GT)],
            stage.at[buf],
            sems[buf],
        ).wait()

    def compute(t, buf):
        l = jnp.bitwise_and(idx2d[0, 0, pl.ds(t * _GT, _GT)], 127)
        tgt = l + lax.mul(lax.iota(jnp.int32, _GT), 128)
        pos = jax.lax.broadcasted_iota(jnp.int32, (_GT, _GT * 128), 1)
        sel = (pos == tgt[:, None]).astype(jnp.float32)
        sv = stage[buf]
        blk = jax.lax.dot_general(
            sv, sel, (((1,), (1,)), ((), ())),
            preferred_element_type=jnp.float32,
            precision=jax.lax.Precision.HIGHEST,
        )
        out_ref[:, t * _GT:(t + 1) * _GT] = blk

    @pl.when(j == 0)
    def _():
        issue(0, 0)

    for t in range(_GPP):
        g_next = j * _GPP + t + 1

        @pl.when(g_next < _N_GT)
        def _():
            issue(g_next, (t + 1) % 2)

        drain(t % 2)
        compute(t, t % 2)


def _tc_gather(idx_tc, table_t):
    grid_spec = pltpu.PrefetchScalarGridSpec(
        num_scalar_prefetch=1,
        grid=(_TC_GRID,),
        in_specs=[
            pl.BlockSpec((1, 1, _GPP * _GT), lambda j, *_: (j, 0, 0)),
            pl.BlockSpec(memory_space=pl.ANY),
        ],
        out_specs=pl.BlockSpec((H_DIM, _GPP * _GT), lambda j, *_: (0, j)),
        scratch_shapes=[
            pltpu.VMEM((2, H_DIM, _GT * 128), jnp.float32),
            pltpu.SemaphoreType.DMA,
            pltpu.SemaphoreType.DMA,
        ],
    )
    return pl.pallas_call(
        _tc_body,
        grid_spec=grid_spec,
        out_shape=jax.ShapeDtypeStruct((H_DIM, _B_TC), jnp.float32),
        compiler_params=pltpu.CompilerParams(
            dimension_semantics=("arbitrary",),
        ),
    )(idx_tc, idx_tc.reshape(_TC_GRID, 1, _GPP * _GT), table_t)


@jax.jit
def kernel(node_id, table):
    idx = jnp.asarray(node_id, jnp.int32)
    table_t = table.T
    sc_out = _sc_gather(idx[:_S_SC], table_t)
    tc_out = _tc_gather(idx[_S_SC:], table_t)
    out_t = jnp.concatenate([sc_out, tc_out], axis=1)
    return out_t.T


# speedup vs baseline: 1.8575x; 1.8575x over previous
import functools

import jax
import jax.numpy as jnp
from jax import lax
from jax.experimental import pallas as pl
from jax.experimental.pallas import tpu as pltpu
from jax.experimental.pallas import tpu_sc as plsc

H_DIM = 64
BATCH = 16384

_S_SC = 12288
_B_TC = BATCH - _S_SC

_NC = 2
_NS = 16
_NW = _NC * _NS
_B_PER_W = _S_SC // _NW
_G = 16
_N_G = _B_PER_W // _G
_HP = 16
_N_HP = H_DIM // _HP

_GT = 16
_N_GT = _B_TC // _GT
_GPP = 8
_TC_GRID = _N_GT // _GPP
_NQ = 8


def _emb_body(idx_hbm, table_hbm, out_hbm, idx_v, staged_v, out_v, sem0, sem1):
    wid = lax.axis_index("s") * _NC + lax.axis_index("c")
    base = wid * _B_PER_W
    pltpu.sync_copy(idx_hbm.at[pl.ds(base, _B_PER_W)], idx_v)

    lanes = lax.iota(jnp.int32, 16)
    sems = (sem0, sem1)

    def starts_of(g):
        r = idx_v[pl.ds(g * _G, _G)]
        a128 = lax.shift_left(lax.shift_right_logical(r, 7), 7)
        return [pl.multiple_of(a128[k], 128) for k in range(_G)]

    def issue(starts, h, buf):
        for k in range(_G):
            pltpu.async_copy(
                table_hbm.at[pl.ds(h * _HP, _HP), pl.ds(starts[k], 128)],
                staged_v.at[buf, k],
                sems[buf],
            )

    def drain(buf):
        for _ in range(_G):
            pltpu.make_async_copy(
                table_hbm.at[pl.ds(0, _HP), pl.ds(0, 128)],
                staged_v.at[buf, 0],
                sems[buf],
            ).wait()

    def extract(l128, g, h, buf):
        for cc in range(_HP):
            vals = plsc.load_gather(
                staged_v.at[buf], [lanes, jnp.full((16,), cc, jnp.int32), l128]
            )
            out_v[h * _HP + cc, pl.ds(g * _G, _G)] = vals

    issue(starts_of(0), 0, 0)

    def _group(g, carry):
        r = idx_v[pl.ds(g * _G, _G)]
        l128 = lax.bitwise_and(r, 127)
        starts = starts_of(g)
        issue(starts, 1, 1)
        drain(0)
        extract(l128, g, 0, 0)
        issue(starts, 2, 0)
        drain(1)
        extract(l128, g, 1, 1)
        issue(starts, 3, 1)
        drain(0)
        extract(l128, g, 2, 0)
        g_next = lax.min(g + 1, _N_G - 1)
        issue(starts_of(g_next), 0, 0)
        drain(1)
        extract(l128, g, 3, 1)
        return carry

    lax.fori_loop(0, _N_G, _group, 0)
    drain(0)
    pltpu.sync_copy(out_v, out_hbm.at[:, pl.ds(base, _B_PER_W)])


def _sc_gather(idx_sc, table_t):
    f = functools.partial(
        pl.kernel,
        mesh=plsc.VectorSubcoreMesh(core_axis_name="c", subcore_axis_name="s"),
        out_type=jax.ShapeDtypeStruct((H_DIM, _S_SC), jnp.float32),
        compiler_params=pltpu.CompilerParams(needs_layout_passes=False),
        scratch_types=[
            pltpu.VMEM((_B_PER_W,), jnp.int32),
            pltpu.VMEM((2, _G, _HP, 128), jnp.float32),
            pltpu.VMEM((H_DIM, _B_PER_W), jnp.float32),
            pltpu.SemaphoreType.DMA,
            pltpu.SemaphoreType.DMA,
        ],
    )(_emb_body)
    return f(idx_sc, table_t)


def _tc_body(idx_smem, idx2d, table_hbm, out_ref, stage, sems):
    j = pl.program_id(0)

    def issue(g, buf):
        for k in range(_GT):
            a = lax.shift_left(lax.shift_right_logical(idx_smem[g * _GT + k], 7), 7)
            pltpu.make_async_copy(
                table_hbm.at[:, pl.ds(pl.multiple_of(a, 128), 128)],
                stage.at[buf, :, pl.ds(k * 128, 128)],
                sems.at[buf, k % _NQ],
            ).start()

    def drain(buf):
        for q in range(_NQ):
            pltpu.make_async_copy(
                table_hbm.at[:, pl.ds(0, (_GT // _NQ) * 128)],
                stage.at[buf, :, pl.ds(0, (_GT // _NQ) * 128)],
                sems.at[buf, q],
            ).wait()

    def compute(t, buf):
        l = jnp.bitwise_and(idx2d[0, 0, pl.ds(t * _GT, _GT)], 127)
        tgt = l + lax.mul(lax.iota(jnp.int32, _GT), 128)
        pos = jax.lax.broadcasted_iota(jnp.int32, (_GT, _GT * 128), 1)
        sel = (pos == tgt[:, None]).astype(jnp.float32)
        sv = stage[buf]
        blk = jax.lax.dot_general(
            sv, sel, (((1,), (1,)), ((), ())),
            preferred_element_type=jnp.float32,
            precision=jax.lax.Precision.HIGHEST,
        )
        out_ref[:, t * _GT:(t + 1) * _GT] = blk

    @pl.when(j == 0)
    def _():
        issue(0, 0)

    for t in range(_GPP):
        g_next = j * _GPP + t + 1

        @pl.when(g_next < _N_GT)
        def _():
            issue(g_next, (t + 1) % 2)

        drain(t % 2)
        compute(t, t % 2)


def _tc_gather(idx_tc, table_t):
    grid_spec = pltpu.PrefetchScalarGridSpec(
        num_scalar_prefetch=1,
        grid=(_TC_GRID,),
        in_specs=[
            pl.BlockSpec((1, 1, _GPP * _GT), lambda j, *_: (j, 0, 0)),
            pl.BlockSpec(memory_space=pl.ANY),
        ],
        out_specs=pl.BlockSpec((H_DIM, _GPP * _GT), lambda j, *_: (0, j)),
        scratch_shapes=[
            pltpu.VMEM((2, H_DIM, _GT * 128), jnp.float32),
            pltpu.SemaphoreType.DMA((2, _NQ)),
        ],
    )
    return pl.pallas_call(
        _tc_body,
        grid_spec=grid_spec,
        out_shape=jax.ShapeDtypeStruct((H_DIM, _B_TC), jnp.float32),
        compiler_params=pltpu.CompilerParams(
            dimension_semantics=("arbitrary",),
        ),
    )(idx_tc, idx_tc.reshape(_TC_GRID, 1, _GPP * _GT), table_t)


@jax.jit
def kernel(node_id, table):
    idx = jnp.asarray(node_id, jnp.int32)
    table_t = table.T
    sc_out = _sc_gather(idx[:_S_SC], table_t)
    tc_out = _tc_gather(idx[_S_SC:], table_t)
    out_t = jnp.concatenate([sc_out, tc_out], axis=1)
    return out_t.T


# speedup vs baseline: 1.9688x; 1.0599x over previous
import functools

import jax
import jax.numpy as jnp
from jax import lax
from jax.experimental import pallas as pl
from jax.experimental.pallas import tpu as pltpu
from jax.experimental.pallas import tpu_sc as plsc

H_DIM = 64
BATCH = 16384

_S_SC = 12288
_B_TC = BATCH - _S_SC

_NC = 2
_NS = 16
_NW = _NC * _NS
_B_PER_W = _S_SC // _NW
_G = 16
_N_G = _B_PER_W // _G
_HP = 16
_N_HP = H_DIM // _HP

_GT = 16
_N_GT = _B_TC // _GT
_GPP = 8
_TC_GRID = _N_GT // _GPP
_NQ = 8
_NB = 4


def _emb_body(idx_hbm, table_hbm, out_hbm, idx_v, staged_v, out_v, sem0, sem1):
    wid = lax.axis_index("s") * _NC + lax.axis_index("c")
    base = wid * _B_PER_W
    pltpu.sync_copy(idx_hbm.at[pl.ds(base, _B_PER_W)], idx_v)

    lanes = lax.iota(jnp.int32, 16)
    sems = (sem0, sem1)

    def starts_of(g):
        r = idx_v[pl.ds(g * _G, _G)]
        a128 = lax.shift_left(lax.shift_right_logical(r, 7), 7)
        return [pl.multiple_of(a128[k], 128) for k in range(_G)]

    def issue(starts, h, buf):
        for k in range(_G):
            pltpu.async_copy(
                table_hbm.at[pl.ds(h * _HP, _HP), pl.ds(starts[k], 128)],
                staged_v.at[buf, k],
                sems[buf],
            )

    def drain(buf):
        for _ in range(_G):
            pltpu.make_async_copy(
                table_hbm.at[pl.ds(0, _HP), pl.ds(0, 128)],
                staged_v.at[buf, 0],
                sems[buf],
            ).wait()

    def extract(l128, g, h, buf):
        for cc in range(_HP):
            vals = plsc.load_gather(
                staged_v.at[buf], [lanes, jnp.full((16,), cc, jnp.int32), l128]
            )
            out_v[h * _HP + cc, pl.ds(g * _G, _G)] = vals

    issue(starts_of(0), 0, 0)

    def _group(g, carry):
        r = idx_v[pl.ds(g * _G, _G)]
        l128 = lax.bitwise_and(r, 127)
        starts = starts_of(g)
        issue(starts, 1, 1)
        drain(0)
        extract(l128, g, 0, 0)
        issue(starts, 2, 0)
        drain(1)
        extract(l128, g, 1, 1)
        issue(starts, 3, 1)
        drain(0)
        extract(l128, g, 2, 0)
        g_next = lax.min(g + 1, _N_G - 1)
        issue(starts_of(g_next), 0, 0)
        drain(1)
        extract(l128, g, 3, 1)
        return carry

    lax.fori_loop(0, _N_G, _group, 0)
    drain(0)
    pltpu.sync_copy(out_v, out_hbm.at[:, pl.ds(base, _B_PER_W)])


def _sc_gather(idx_sc, table_t):
    f = functools.partial(
        pl.kernel,
        mesh=plsc.VectorSubcoreMesh(core_axis_name="c", subcore_axis_name="s"),
        out_type=jax.ShapeDtypeStruct((H_DIM, _S_SC), jnp.float32),
        compiler_params=pltpu.CompilerParams(needs_layout_passes=False),
        scratch_types=[
            pltpu.VMEM((_B_PER_W,), jnp.int32),
            pltpu.VMEM((2, _G, _HP, 128), jnp.float32),
            pltpu.VMEM((H_DIM, _B_PER_W), jnp.float32),
            pltpu.SemaphoreType.DMA,
            pltpu.SemaphoreType.DMA,
        ],
    )(_emb_body)
    return f(idx_sc, table_t)


def _tc_body(idx_smem, idx2d, table_hbm, out_ref, stage, sems):
    j = pl.program_id(0)

    def issue(g, buf):
        for k in range(_GT):
            a = lax.shift_left(lax.shift_right_logical(idx_smem[g * _GT + k], 7), 7)
            pltpu.make_async_copy(
                table_hbm.at[:, pl.ds(pl.multiple_of(a, 128), 128)],
                stage.at[buf, :, pl.ds(k * 128, 128)],
                sems.at[buf, k % _NQ],
            ).start()

    def drain(buf):
        for q in range(_NQ):
            pltpu.make_async_copy(
                table_hbm.at[:, pl.ds(0, (_GT // _NQ) * 128)],
                stage.at[buf, :, pl.ds(0, (_GT // _NQ) * 128)],
                sems.at[buf, q],
            ).wait()

    def compute(t, buf):
        l = jnp.bitwise_and(idx2d[0, 0, pl.ds(t * _GT, _GT)], 127)
        tgt = l + lax.mul(lax.iota(jnp.int32, _GT), 128)
        pos = jax.lax.broadcasted_iota(jnp.int32, (_GT, _GT * 128), 1)
        sel = (pos == tgt[:, None]).astype(jnp.float32)
        sv = stage[buf]
        blk = jax.lax.dot_general(
            sv, sel, (((1,), (1,)), ((), ())),
            preferred_element_type=jnp.float32,
            precision=jax.lax.Precision.HIGHEST,
        )
        out_ref[:, t * _GT:(t + 1) * _GT] = blk

    @pl.when(j == 0)
    def _():
        for b in range(_NB - 1):
            issue(b, b)

    for t in range(_GPP):
        g_next = j * _GPP + t + (_NB - 1)

        @pl.when(g_next < _N_GT)
        def _():
            issue(g_next, (t + _NB - 1) % _NB)

        drain(t % _NB)
        compute(t, t % _NB)


def _tc_gather(idx_tc, table_t):
    grid_spec = pltpu.PrefetchScalarGridSpec(
        num_scalar_prefetch=1,
        grid=(_TC_GRID,),
        in_specs=[
            pl.BlockSpec((1, 1, _GPP * _GT), lambda j, *_: (j, 0, 0)),
            pl.BlockSpec(memory_space=pl.ANY),
        ],
        out_specs=pl.BlockSpec((H_DIM, _GPP * _GT), lambda j, *_: (0, j)),
        scratch_shapes=[
            pltpu.VMEM((_NB, H_DIM, _GT * 128), jnp.float32),
            pltpu.SemaphoreType.DMA((_NB, _NQ)),
        ],
    )
    return pl.pallas_call(
        _tc_body,
        grid_spec=grid_spec,
        out_shape=jax.ShapeDtypeStruct((H_DIM, _B_TC), jnp.float32),
        compiler_params=pltpu.CompilerParams(
            dimension_semantics=("arbitrary",),
        ),
    )(idx_tc, idx_tc.reshape(_TC_GRID, 1, _GPP * _GT), table_t)


@jax.jit
def kernel(node_id, table):
    idx = jnp.asarray(node_id, jnp.int32)
    table_t = table.T
    sc_out = _sc_gather(idx[:_S_SC], table_t)
    tc_out = _tc_gather(idx[_S_SC:], table_t)
    out_t = jnp.concatenate([sc_out, tc_out], axis=1)
    return out_t.T


# speedup vs baseline: 2.5724x; 1.3066x over previous
import functools

import jax
import jax.numpy as jnp
from jax import lax
from jax.experimental import pallas as pl
from jax.experimental.pallas import tpu as pltpu
from jax.experimental.pallas import tpu_sc as plsc

H_DIM = 64
BATCH = 16384
_NC = 2
_NS = 16
_NW = _NC * _NS
_B_PER_W = BATCH // _NW
_G = 16
_N_G = _B_PER_W // _G
_HP = 16
_N_HP = H_DIM // _HP


def _emb_body(idx_hbm, table_hbm, out_hbm, idx_v, staged_v, out_v, sem0, sem1):
    wid = lax.axis_index("s") * _NC + lax.axis_index("c")
    base = wid * _B_PER_W
    pltpu.sync_copy(idx_hbm.at[pl.ds(base, _B_PER_W)], idx_v)

    lanes = lax.iota(jnp.int32, 16)
    sems = (sem0, sem1)

    def starts_of(g):
        r = idx_v[pl.ds(g * _G, _G)]
        a128 = lax.shift_left(lax.shift_right_logical(r, 7), 7)
        return [pl.multiple_of(a128[k], 128) for k in range(_G)]

    def issue(starts, h, buf):
        for k in range(_G):
            pltpu.async_copy(
                table_hbm.at[pl.ds(h * _HP, _HP), pl.ds(starts[k], 128)],
                staged_v.at[buf, k],
                sems[buf],
            )

    def drain(buf):
        for _ in range(_G):
            pltpu.make_async_copy(
                table_hbm.at[pl.ds(0, _HP), pl.ds(0, 128)],
                staged_v.at[buf, 0],
                sems[buf],
            ).wait()

    def extract(l128, g, h, buf):
        for cc in range(_HP):
            vals = plsc.load_gather(
                staged_v.at[buf], [lanes, jnp.full((16,), cc, jnp.int32), l128]
            )
            out_v[h * _HP + cc, pl.ds(g * _G, _G)] = vals

    issue(starts_of(0), 0, 0)

    def _group(g, carry):
        r = idx_v[pl.ds(g * _G, _G)]
        l128 = lax.bitwise_and(r, 127)
        starts = starts_of(g)
        issue(starts, 1, 1)
        drain(0)
        extract(l128, g, 0, 0)
        issue(starts, 2, 0)
        drain(1)
        extract(l128, g, 1, 1)
        issue(starts, 3, 1)
        drain(0)
        extract(l128, g, 2, 0)
        g_next = lax.min(g + 1, _N_G - 1)
        issue(starts_of(g_next), 0, 0)
        drain(1)
        extract(l128, g, 3, 1)
        return carry

    lax.fori_loop(0, _N_G, _group, 0)
    drain(0)
    pltpu.sync_copy(out_v, out_hbm.at[:, pl.ds(base, _B_PER_W)])


@jax.jit
def kernel(node_id, table):
    idx = jnp.asarray(node_id, jnp.int32)
    table_t = table.T
    f = functools.partial(
        pl.kernel,
        mesh=plsc.VectorSubcoreMesh(core_axis_name="c", subcore_axis_name="s"),
        out_type=jax.ShapeDtypeStruct((H_DIM, BATCH), jnp.float32),
        compiler_params=pltpu.CompilerParams(needs_layout_passes=False),
        scratch_types=[
            pltpu.VMEM((_B_PER_W,), jnp.int32),
            pltpu.VMEM((2, _G, _HP, 128), jnp.float32),
            pltpu.VMEM((H_DIM, _B_PER_W), jnp.float32),
            pltpu.SemaphoreType.DMA,
            pltpu.SemaphoreType.DMA,
        ],
    )(_emb_body)
    out_t = f(idx, table_t)
    return out_t.T
